# baseline (speedup 1.0000x reference)
import functools

import jax
import jax.numpy as jnp
from jax import lax
from jax.experimental import pallas as pl
from jax.experimental.pallas import tpu as pltpu
from jax.experimental.pallas import tpu_sc as plsc

N_NODES = 10000
FEAT = 128
E_EDGES = 320000
NC = 2
NS = 16
CHUNK = 128
E_PAD = 327680
CPT16 = E_PAD // NS // CHUNK
CPT32 = E_PAD // (NC * NS) // CHUNK
IB_SP = 16
IB_DG = 16
NP = 10112
RPT = NP // NS
BN = 2000

_mesh = plsc.VectorSubcoreMesh(core_axis_name="c", subcore_axis_name="s")


def _deg_body(row2d, ones128, zeros128, out0, out1, rix, buf, acc, sem):
    c = lax.axis_index("c")
    s = lax.axis_index("s")
    wb = s * RPT
    pltpu.sync_copy(zeros128, buf)
    for k in range(4):
        pltpu.sync_copy(buf, acc.at[pl.ds(wb + k * CHUNK, CHUNK)])
    pltpu.sync_copy(buf.at[pl.ds(0, RPT - 4 * CHUNK)],
                    acc.at[pl.ds(wb + 4 * CHUNK, RPT - 4 * CHUNK)])
    pltpu.sync_copy(ones128, buf)
    wid = s * NC + c
    plsc.subcore_barrier()

    def blk(b, carry):
        pltpu.sync_copy(row2d.at[pl.ds(wid * CPT32 + b * IB_DG, IB_DG)], rix)

        def body(i, c2):
            pltpu.async_copy(buf, acc.at[rix.at[i]], sem, add=True)
            return c2

        lax.fori_loop(0, IB_DG, body, 0)

        def drain(i, c2):
            pltpu.make_async_copy(buf, acc.at[rix.at[0]], sem).wait()
            return c2

        lax.fori_loop(0, IB_DG, drain, 0)
        return carry

    lax.fori_loop(0, CPT32 // IB_DG, blk, 0)
    plsc.subcore_barrier()

    @pl.when(c == 0)
    def _():
        pltpu.sync_copy(acc.at[pl.ds(wb, RPT)], out0.at[pl.ds(wb, RPT)])

    @pl.when(c == 1)
    def _():
        pltpu.sync_copy(acc.at[pl.ds(wb, RPT)], out1.at[pl.ds(wb, RPT)])


def _spmm_body(hx, hn, row2d, col2d, zeros128, ox0, ox1, on0, on1,
               cix, rix, rows0, rows1, acc, g0, g1):
    c = lax.axis_index("c")
    s = lax.axis_index("s")
    wb = s * RPT
    wid = s * NC + c

    def zero_acc():
        pltpu.sync_copy(zeros128, rows0)
        for k in range(4):
            pltpu.sync_copy(rows0, acc.at[pl.ds(wb + k * CHUNK, CHUNK)])
        pltpu.sync_copy(rows0.at[pl.ds(0, RPT - 4 * CHUNK)],
                        acc.at[pl.ds(wb + 4 * CHUNK, RPT - 4 * CHUNK)])

    def run(h_hbm):
        H = CHUNK // 2

        def gat(i, dst, sem):
            pltpu.async_copy(h_hbm.at[cix.at[i, pl.ds(0, H)]],
                             dst.at[pl.ds(0, H)], sem)
            pltpu.async_copy(h_hbm.at[cix.at[i, pl.ds(H, H)]],
                             dst.at[pl.ds(H, H)], sem)

        def wat(dst, sem):
            pltpu.make_async_copy(h_hbm.at[cix.at[0, pl.ds(0, H)]],
                                  dst.at[pl.ds(0, H)], sem).wait()
            pltpu.make_async_copy(h_hbm.at[cix.at[0, pl.ds(0, H)]],
                                  dst.at[pl.ds(H, H)], sem).wait()

        def blk(b, carry):
            base = wid * CPT32 + b * IB_SP
            pltpu.sync_copy(col2d.at[pl.ds(base, IB_SP)], cix)
            pltpu.sync_copy(row2d.at[pl.ds(base, IB_SP)], rix)
            gat(0, rows0, g0)

            def body(j, c2):
                i0 = 2 * j
                gat(i0 + 1, rows1, g1)
                wat(rows0, g0)
                pltpu.sync_copy(rows0, acc.at[rix.at[i0]], add=True)

                @pl.when(j < IB_SP // 2 - 1)
                def _():
                    gat(i0 + 2, rows0, g0)

                wat(rows1, g1)
                pltpu.sync_copy(rows1, acc.at[rix.at[i0 + 1]], add=True)
                return c2

            lax.fori_loop(0, IB_SP // 2, body, 0)
            return carry

        lax.fori_loop(0, CPT32 // IB_SP, blk, 0)

    def writeback(o0, o1):
        @pl.when(c == 0)
        def _():
            pltpu.sync_copy(acc.at[pl.ds(wb, RPT)], o0.at[pl.ds(wb, RPT)])

        @pl.when(c == 1)
        def _():
            pltpu.sync_copy(acc.at[pl.ds(wb, RPT)], o1.at[pl.ds(wb, RPT)])

    zero_acc()
    plsc.subcore_barrier()
    run(hx)
    plsc.subcore_barrier()
    writeback(ox0, ox1)
    zero_acc()
    plsc.subcore_barrier()
    run(hn)
    plsc.subcore_barrier()
    writeback(on0, on1)


def _make_deg_kernel(interpret=False):
    return pl.kernel(
        _deg_body,
        out_type=(jax.ShapeDtypeStruct((NP, FEAT), jnp.float32),
                  jax.ShapeDtypeStruct((NP, FEAT), jnp.float32)),
        mesh=_mesh,
        scratch_types=[
            pltpu.VMEM((IB_DG, CHUNK), jnp.int32),
            pltpu.VMEM((CHUNK, FEAT), jnp.float32),
            pltpu.VMEM_SHARED((NP, FEAT), jnp.float32),
            pltpu.SemaphoreType.DMA,
        ],
        interpret=interpret,
    )


def _make_spmm_kernel(interpret=False):
    return pl.kernel(
        _spmm_body,
        out_type=(jax.ShapeDtypeStruct((NP, FEAT), jnp.float32),
                  jax.ShapeDtypeStruct((NP, FEAT), jnp.float32),
                  jax.ShapeDtypeStruct((NP, FEAT), jnp.float32),
                  jax.ShapeDtypeStruct((NP, FEAT), jnp.float32)),
        mesh=_mesh,
        scratch_types=[
            pltpu.VMEM((IB_SP, CHUNK), jnp.int32),
            pltpu.VMEM((IB_SP, CHUNK), jnp.int32),
            pltpu.VMEM((CHUNK, FEAT), jnp.float32),
            pltpu.VMEM((CHUNK, FEAT), jnp.float32),
            pltpu.VMEM_SHARED((NP, FEAT), jnp.float32),
            pltpu.SemaphoreType.DMA,
            pltpu.SemaphoreType.DMA,
        ],
        interpret=interpret,
    )


_deg_kernel = _make_deg_kernel()
_spmm_kernel = _make_spmm_kernel()


def _k1_body(d0, d1, x, sf, ox, on, od):
    deg = d0[:, 0:1] + d1[:, 0:1] + 1.0
    dinv = lax.rsqrt(deg)
    od[...] = jnp.broadcast_to(dinv, (BN, FEAT))
    ox[...] = x[...] * dinv
    on[...] = sf[...] * dinv


def _k2_body(dv, x0, x1, n0, n1, h1x, h1n, ox, on):
    d2 = dv[...] * dv[...]
    ox[...] = (x0[...] + x1[...] + h1x[...]) * d2
    on[...] = (n0[...] + n1[...] + h1n[...]) * d2


def _k3_body(dv, x0, x1, n0, n1, h2x, h2n, wsgc, bsgc, wl1, wl2, bl, out):
    dinv = dv[...]
    hx = (x0[...] + x1[...] + h2x[...]) * dinv
    noise = (n0[...] + n1[...] + h2n[...]) * dinv
    z = jnp.dot(hx, wsgc[...], preferred_element_type=jnp.float32) + bsgc[...]
    nrm = jnp.sqrt(jnp.sum(z * z, axis=-1, keepdims=True))
    z = z / jnp.maximum(nrm, 1e-12)
    out[...] = (jnp.dot(z, wl1[...], preferred_element_type=jnp.float32)
                + jnp.dot(noise, wl2[...], preferred_element_type=jnp.float32)
                + bl[...])


def _row_spec(w):
    return pl.BlockSpec((BN, w), lambda i: (i, 0))


def _full_spec(r, w):
    return pl.BlockSpec((r, w), lambda i: (0, 0))


_GRID = N_NODES // BN

_k1 = pl.pallas_call(
    _k1_body,
    grid=(_GRID,),
    in_specs=[_row_spec(FEAT), _row_spec(FEAT), _row_spec(FEAT), _row_spec(FEAT)],
    out_specs=(_row_spec(FEAT), _row_spec(FEAT), _row_spec(FEAT)),
    out_shape=(jax.ShapeDtypeStruct((N_NODES, FEAT), jnp.float32),
               jax.ShapeDtypeStruct((N_NODES, FEAT), jnp.float32),
               jax.ShapeDtypeStruct((N_NODES, FEAT), jnp.float32)),
)

_k2 = pl.pallas_call(
    _k2_body,
    grid=(_GRID,),
    in_specs=[_row_spec(FEAT), _row_spec(FEAT), _row_spec(FEAT),
              _row_spec(FEAT), _row_spec(FEAT), _row_spec(FEAT), _row_spec(FEAT)],
    out_specs=(_row_spec(FEAT), _row_spec(FEAT)),
    out_shape=(jax.ShapeDtypeStruct((N_NODES, FEAT), jnp.float32),
               jax.ShapeDtypeStruct((N_NODES, FEAT), jnp.float32)),
)

_k3 = pl.pallas_call(
    _k3_body,
    grid=(_GRID,),
    in_specs=[_row_spec(FEAT), _row_spec(FEAT), _row_spec(FEAT),
              _row_spec(FEAT), _row_spec(FEAT), _row_spec(FEAT), _row_spec(FEAT),
              _full_spec(FEAT, FEAT), _full_spec(1, FEAT),
              _full_spec(FEAT, FEAT), _full_spec(FEAT, FEAT), _full_spec(1, FEAT)],
    out_specs=pl.BlockSpec((BN, FEAT), lambda i: (i, 0)),
    out_shape=jax.ShapeDtypeStruct((N_NODES, FEAT), jnp.float32),
)


@jax.jit
def kernel(x, edge_index, stochastic_feature, W_sgc, b_sgc, W_last, b_last):
    row = edge_index[0].astype(jnp.int32)
    col = edge_index[1].astype(jnp.int32)
    pad = E_PAD - E_EDGES
    rowp = jnp.concatenate([row, jnp.full((pad,), N_NODES, jnp.int32)])
    colp = jnp.concatenate([col, jnp.zeros((pad,), jnp.int32)])
    row2d = rowp.reshape(E_PAD // CHUNK, CHUNK)
    col2d = colp.reshape(E_PAD // CHUNK, CHUNK)
    ones128 = jnp.ones((CHUNK, FEAT), jnp.float32)
    zeros128 = jnp.zeros((CHUNK, FEAT), jnp.float32)

    d0, d1 = _deg_kernel(row2d, ones128, zeros128)

    h1x, h1n, dv = _k1(d0[:N_NODES], d1[:N_NODES], x, stochastic_feature)
    x0, x1, n0, n1 = _spmm_kernel(h1x, h1n, row2d, col2d, zeros128)
    h2x, h2n = _k2(dv, x0[:N_NODES], x1[:N_NODES], n0[:N_NODES], n1[:N_NODES],
                   h1x, h1n)
    y0, y1, m0, m1 = _spmm_kernel(h2x, h2n, row2d, col2d, zeros128)
    out = _k3(dv, y0[:N_NODES], y1[:N_NODES], m0[:N_NODES], m1[:N_NODES],
              h2x, h2n,
              W_sgc, b_sgc.reshape(1, FEAT),
              W_last[:FEAT], W_last[FEAT:], b_last.reshape(1, FEAT))
    return out


# speedup vs baseline: 1.4984x; 1.4984x over previous
import functools

import jax
import jax.numpy as jnp
from jax import lax
from jax.experimental import pallas as pl
from jax.experimental.pallas import tpu as pltpu
from jax.experimental.pallas import tpu_sc as plsc

N_NODES = 10000
FEAT = 128
E_EDGES = 320000
NC = 2
NS = 16
CHUNK = 128
E_PAD = 327680
CPT16 = E_PAD // NS // CHUNK
CPT32 = E_PAD // (NC * NS) // CHUNK
IB_SP = 32
IB_DG = 16
NP = 10112
RPT = NP // NS
BN = 2000

_mesh = plsc.VectorSubcoreMesh(core_axis_name="c", subcore_axis_name="s")


def _deg_body(row2d, ones128, zeros128, out0, out1, rix, buf, acc, sem):
    c = lax.axis_index("c")
    s = lax.axis_index("s")
    wb = s * RPT
    pltpu.sync_copy(zeros128, buf)
    for k in range(4):
        pltpu.sync_copy(buf, acc.at[pl.ds(wb + k * CHUNK, CHUNK)])
    pltpu.sync_copy(buf.at[pl.ds(0, RPT - 4 * CHUNK)],
                    acc.at[pl.ds(wb + 4 * CHUNK, RPT - 4 * CHUNK)])
    pltpu.sync_copy(ones128, buf)
    wid = s * NC + c
    plsc.subcore_barrier()

    def blk(b, carry):
        pltpu.sync_copy(row2d.at[pl.ds(wid * CPT32 + b * IB_DG, IB_DG)], rix)

        def body(i, c2):
            pltpu.async_copy(buf, acc.at[rix.at[i]], sem, add=True)
            return c2

        lax.fori_loop(0, IB_DG, body, 0)

        def drain(i, c2):
            pltpu.make_async_copy(buf, acc.at[rix.at[0]], sem).wait()
            return c2

        lax.fori_loop(0, IB_DG, drain, 0)
        return carry

    lax.fori_loop(0, CPT32 // IB_DG, blk, 0)
    plsc.subcore_barrier()

    @pl.when(c == 0)
    def _():
        pltpu.sync_copy(acc.at[pl.ds(wb, RPT)], out0.at[pl.ds(wb, RPT)])

    @pl.when(c == 1)
    def _():
        pltpu.sync_copy(acc.at[pl.ds(wb, RPT)], out1.at[pl.ds(wb, RPT)])


def _spmm_body(hx, hn, row2d, col2d, zeros128, outx, outn,
               cix, rix, rows0, rows1, acc, g0, g1):
    c = lax.axis_index("c")
    s = lax.axis_index("s")
    wb = s * RPT
    pltpu.sync_copy(zeros128, rows0)
    for k in range(4):
        pltpu.sync_copy(rows0, acc.at[pl.ds(wb + k * CHUNK, CHUNK)])
    pltpu.sync_copy(rows0.at[pl.ds(0, RPT - 4 * CHUNK)],
                    acc.at[pl.ds(wb + 4 * CHUNK, RPT - 4 * CHUNK)])
    plsc.subcore_barrier()

    def run(h_hbm):
        H = CHUNK // 2

        def gat(i, dst, sem):
            pltpu.async_copy(h_hbm.at[cix.at[i, pl.ds(0, H)]],
                             dst.at[pl.ds(0, H)], sem)
            pltpu.async_copy(h_hbm.at[cix.at[i, pl.ds(H, H)]],
                             dst.at[pl.ds(H, H)], sem)

        def wat(dst, sem):
            pltpu.make_async_copy(h_hbm.at[cix.at[0, pl.ds(0, H)]],
                                  dst.at[pl.ds(0, H)], sem).wait()
            pltpu.make_async_copy(h_hbm.at[cix.at[0, pl.ds(0, H)]],
                                  dst.at[pl.ds(H, H)], sem).wait()

        def blk(b, carry):
            base = s * CPT16 + b * IB_SP
            pltpu.sync_copy(col2d.at[pl.ds(base, IB_SP)], cix)
            pltpu.sync_copy(row2d.at[pl.ds(base, IB_SP)], rix)
            gat(0, rows0, g0)

            def body(j, c2):
                i0 = 2 * j
                gat(i0 + 1, rows1, g1)
                wat(rows0, g0)
                pltpu.sync_copy(rows0, acc.at[rix.at[i0]], add=True)

                @pl.when(j < IB_SP // 2 - 1)
                def _():
                    gat(i0 + 2, rows0, g0)

                wat(rows1, g1)
                pltpu.sync_copy(rows1, acc.at[rix.at[i0 + 1]], add=True)
                return c2

            lax.fori_loop(0, IB_SP // 2, body, 0)
            return carry

        lax.fori_loop(0, CPT16 // IB_SP, blk, 0)

    @pl.when(c == 0)
    def _():
        run(hx)

    @pl.when(c == 1)
    def _():
        run(hn)

    plsc.subcore_barrier()

    @pl.when(c == 0)
    def _():
        pltpu.sync_copy(acc.at[pl.ds(wb, RPT)], outx.at[pl.ds(wb, RPT)])

    @pl.when(c == 1)
    def _():
        pltpu.sync_copy(acc.at[pl.ds(wb, RPT)], outn.at[pl.ds(wb, RPT)])


def _make_deg_kernel(interpret=False):
    return pl.kernel(
        _deg_body,
        out_type=(jax.ShapeDtypeStruct((NP, FEAT), jnp.float32),
                  jax.ShapeDtypeStruct((NP, FEAT), jnp.float32)),
        mesh=_mesh,
        scratch_types=[
            pltpu.VMEM((IB_DG, CHUNK), jnp.int32),
            pltpu.VMEM((CHUNK, FEAT), jnp.float32),
            pltpu.VMEM_SHARED((NP, FEAT), jnp.float32),
            pltpu.SemaphoreType.DMA,
        ],
        interpret=interpret,
    )


def _make_spmm_kernel(interpret=False):
    return pl.kernel(
        _spmm_body,
        out_type=(jax.ShapeDtypeStruct((NP, FEAT), jnp.float32),
                  jax.ShapeDtypeStruct((NP, FEAT), jnp.float32)),
        mesh=_mesh,
        scratch_types=[
            pltpu.VMEM((IB_SP, CHUNK), jnp.int32),
            pltpu.VMEM((IB_SP, CHUNK), jnp.int32),
            pltpu.VMEM((CHUNK, FEAT), jnp.float32),
            pltpu.VMEM((CHUNK, FEAT), jnp.float32),
            pltpu.VMEM_SHARED((NP, FEAT), jnp.float32),
            pltpu.SemaphoreType.DMA,
            pltpu.SemaphoreType.DMA,
        ],
        interpret=interpret,
    )


_deg_kernel = _make_deg_kernel()
_spmm_kernel = _make_spmm_kernel()


def _k1_body(d0, d1, x, sf, ox, on, od):
    deg = d0[:, 0:1] + d1[:, 0:1] + 1.0
    dinv = lax.rsqrt(deg)
    od[...] = jnp.broadcast_to(dinv, (BN, FEAT))
    ox[...] = x[...] * dinv
    on[...] = sf[...] * dinv


def _k2_body(dv, s1x, s1n, h1x, h1n, ox, on):
    d2 = dv[...] * dv[...]
    ox[...] = (s1x[...] + h1x[...]) * d2
    on[...] = (s1n[...] + h1n[...]) * d2


def _k3_body(dv, s2x, s2n, h2x, h2n, wsgc, bsgc, wl1, wl2, bl, out):
    dinv = dv[...]
    hx = (s2x[...] + h2x[...]) * dinv
    noise = (s2n[...] + h2n[...]) * dinv
    z = jnp.dot(hx, wsgc[...], preferred_element_type=jnp.float32) + bsgc[...]
    nrm = jnp.sqrt(jnp.sum(z * z, axis=-1, keepdims=True))
    z = z / jnp.maximum(nrm, 1e-12)
    out[...] = (jnp.dot(z, wl1[...], preferred_element_type=jnp.float32)
                + jnp.dot(noise, wl2[...], preferred_element_type=jnp.float32)
                + bl[...])


def _row_spec(w):
    return pl.BlockSpec((BN, w), lambda i: (i, 0))


def _full_spec(r, w):
    return pl.BlockSpec((r, w), lambda i: (0, 0))


_GRID = N_NODES // BN

_k1 = pl.pallas_call(
    _k1_body,
    grid=(_GRID,),
    in_specs=[_row_spec(FEAT), _row_spec(FEAT), _row_spec(FEAT), _row_spec(FEAT)],
    out_specs=(_row_spec(FEAT), _row_spec(FEAT), _row_spec(FEAT)),
    out_shape=(jax.ShapeDtypeStruct((N_NODES, FEAT), jnp.float32),
               jax.ShapeDtypeStruct((N_NODES, FEAT), jnp.float32),
               jax.ShapeDtypeStruct((N_NODES, FEAT), jnp.float32)),
)

_k2 = pl.pallas_call(
    _k2_body,
    grid=(_GRID,),
    in_specs=[_row_spec(FEAT),
              _row_spec(FEAT), _row_spec(FEAT), _row_spec(FEAT), _row_spec(FEAT)],
    out_specs=(_row_spec(FEAT), _row_spec(FEAT)),
    out_shape=(jax.ShapeDtypeStruct((N_NODES, FEAT), jnp.float32),
               jax.ShapeDtypeStruct((N_NODES, FEAT), jnp.float32)),
)

_k3 = pl.pallas_call(
    _k3_body,
    grid=(_GRID,),
    in_specs=[_row_spec(FEAT),
              _row_spec(FEAT), _row_spec(FEAT), _row_spec(FEAT), _row_spec(FEAT),
              _full_spec(FEAT, FEAT), _full_spec(1, FEAT),
              _full_spec(FEAT, FEAT), _full_spec(FEAT, FEAT), _full_spec(1, FEAT)],
    out_specs=pl.BlockSpec((BN, FEAT), lambda i: (i, 0)),
    out_shape=jax.ShapeDtypeStruct((N_NODES, FEAT), jnp.float32),
)


@jax.jit
def kernel(x, edge_index, stochastic_feature, W_sgc, b_sgc, W_last, b_last):
    row = edge_index[0].astype(jnp.int32)
    col = edge_index[1].astype(jnp.int32)
    pad = E_PAD - E_EDGES
    rowp = jnp.concatenate([row, jnp.full((pad,), N_NODES, jnp.int32)])
    colp = jnp.concatenate([col, jnp.zeros((pad,), jnp.int32)])
    row2d = rowp.reshape(E_PAD // CHUNK, CHUNK)
    col2d = colp.reshape(E_PAD // CHUNK, CHUNK)
    ones128 = jnp.ones((CHUNK, FEAT), jnp.float32)
    zeros128 = jnp.zeros((CHUNK, FEAT), jnp.float32)

    d0, d1 = _deg_kernel(row2d, ones128, zeros128)

    h1x, h1n, dv = _k1(d0[:N_NODES], d1[:N_NODES], x, stochastic_feature)
    s1x, s1n = _spmm_kernel(h1x, h1n, row2d, col2d, zeros128)
    h2x, h2n = _k2(dv, s1x[:N_NODES], s1n[:N_NODES], h1x, h1n)
    s2x, s2n = _spmm_kernel(h2x, h2n, row2d, col2d, zeros128)
    out = _k3(dv, s2x[:N_NODES], s2n[:N_NODES], h2x, h2n,
              W_sgc, b_sgc.reshape(1, FEAT),
              W_last[:FEAT], W_last[FEAT:], b_last.reshape(1, FEAT))
    return out
